# baseline (speedup 1.0000x reference)
from functools import partial

import jax
import jax.numpy as jnp
from jax import lax
from jax.experimental import pallas as pl
from jax.experimental.pallas import tpu as pltpu

_EPS = 1e-5


def _bn_cols(st_ref, gb_ref, row, count):
    st = st_ref[...]
    s = jnp.sum(st[:, :, 0:1], axis=0)
    sq = jnp.sum(st[:, :, 1:2], axis=0)
    mean = s / count
    var = jnp.maximum(sq / count - mean * mean, 0.0)
    gamma = gb_ref[2 * row]
    beta = gb_ref[2 * row + 1]
    scale = gamma * lax.rsqrt(var + _EPS)
    shift = beta - mean * scale
    return scale, shift


def _accum(h, s_pl, q_pl):
    if s_pl is None:
        return h, h * h
    return s_pl + h, q_pl + h * h


def _store_stats(st_ref, s_pl, q_pl):
    st_ref[0, :, 0:1] = jnp.sum(s_pl, axis=1, keepdims=True)
    st_ref[0, :, 1:2] = jnp.sum(q_pl, axis=1, keepdims=True)


def _conv1_kernel(x_ref, w1t_ref, h1_ref, st_ref, *, g_imgs):
    f32 = jnp.float32
    s_pl = q_pl = None
    for g in range(g_imgs):
        h1 = jnp.dot(w1t_ref[...], x_ref[g], preferred_element_type=f32)
        h1_ref[g] = h1.astype(jnp.bfloat16)
        s_pl, q_pl = _accum(h1, s_pl, q_pl)
    _store_stats(st_ref, s_pl, q_pl)


def _conv2_kernel(h1_ref, st1_ref, gb_ref, w2_ref, h2_ref, st_ref, *,
                  g_imgs, c, hw, w_img, count):
    f32 = jnp.float32
    bf16 = jnp.bfloat16
    sc, sh = _bn_cols(st1_ref, gb_ref, 0, count)
    scb = sc.astype(bf16)
    shb = sh.astype(bf16)
    ww = lax.broadcasted_iota(jnp.int32, (1, hw), 1) % w_img
    ml = (ww >= 1).astype(f32)
    mr = (ww <= w_img - 2).astype(f32)
    zrow = jnp.zeros((c, w_img), bf16)
    z1 = jnp.zeros((c, 1), f32)
    s_pl = q_pl = None
    for g in range(g_imgs):
        yb = jnp.maximum(h1_ref[g] * scb + shb, 0)
        u_m = jnp.concatenate([zrow, yb[:, :hw - w_img]], axis=1)
        u_p = jnp.concatenate([yb[:, w_img:], zrow], axis=1)
        slab = jnp.concatenate([u_m, yb, u_p], axis=0)
        d_all = jnp.dot(w2_ref[...], slab, preferred_element_type=f32)
        d_m = d_all[0:c]
        d_0 = d_all[c:2 * c]
        d_p = d_all[2 * c:3 * c]
        h2 = (d_0
              + ml * jnp.concatenate([z1, d_m[:, :hw - 1]], axis=1)
              + mr * jnp.concatenate([d_p[:, 1:], z1], axis=1))
        h2_ref[g] = h2.astype(bf16)
        s_pl, q_pl = _accum(h2, s_pl, q_pl)
    _store_stats(st_ref, s_pl, q_pl)


def _conv3_stats_kernel(h2_ref, st2_ref, gb_ref, st_ref, gram_ref, *,
                        g_imgs, count):
    f32 = jnp.float32
    bf16 = jnp.bfloat16
    sc, sh = _bn_cols(st2_ref, gb_ref, 1, count)
    scb = sc.astype(bf16)
    shb = sh.astype(bf16)
    s_pl = g_acc = None
    for g in range(g_imgs):
        y2 = jnp.maximum(h2_ref[g] * scb + shb, 0)
        gr = lax.dot_general(y2, y2, (((1,), (1,)), ((), ())),
                             preferred_element_type=f32)
        s_pl = y2.astype(f32) if s_pl is None else s_pl + y2
        g_acc = gr if g_acc is None else g_acc + gr
    st_ref[0, :, 0:1] = jnp.sum(s_pl, axis=1, keepdims=True)
    gram_ref[0] = g_acc


def _bn3_cols(sy_ref, gram_ref, gb_ref, w3t_ref, count):
    f32 = jnp.float32
    sy = jnp.sum(sy_ref[:, :, 0:1], axis=0)
    gm = jnp.sum(gram_ref[...], axis=0)
    w3f = w3t_ref[...].astype(f32)
    s3 = jnp.dot(w3f, sy, preferred_element_type=f32)
    b = jnp.dot(w3f, gm, preferred_element_type=f32)
    q3 = jnp.sum(b * w3f, axis=1, keepdims=True)
    mean = s3 / count
    var = jnp.maximum(q3 / count - mean * mean, 0.0)
    scale = gb_ref[4] * lax.rsqrt(var + _EPS)
    shift = gb_ref[5] - mean * scale
    return scale, shift


def _final_kernel(h2_ref, x_ref, st2_ref, sy3_ref, gram_ref, gb_ref, w3t_ref,
                  o_ref, *, g_imgs, count):
    f32 = jnp.float32
    sc2, sh2 = _bn_cols(st2_ref, gb_ref, 1, count)
    scb2 = sc2.astype(jnp.bfloat16)
    shb2 = sh2.astype(jnp.bfloat16)
    sc3, sh3 = _bn3_cols(sy3_ref, gram_ref, gb_ref, w3t_ref, count)
    for g in range(g_imgs):
        y2 = jnp.maximum(h2_ref[g] * scb2 + shb2, 0)
        h3 = jnp.dot(w3t_ref[...], y2, preferred_element_type=f32)
        o = jnp.maximum(h3 * sc3 + sh3 + x_ref[g].astype(f32), 0.0)
        o_ref[g] = o.astype(jnp.bfloat16)


def kernel(x_nchw, w1, w2_hwio, w3, gammas, betas):
    N, C, H, W = x_nchw.shape
    planes = w1.shape[1]
    assert planes == C, "residual add requires planes == inplanes"
    HW = H * W
    M = float(N * HW)
    f32 = jnp.float32
    bf16 = jnp.bfloat16

    G = 16 if N % 16 == 0 else (8 if N % 8 == 0 else 1)
    ntiles = N // G

    x3 = x_nchw.reshape(N, C, HW).astype(bf16)

    w1t = jnp.transpose(w1).astype(bf16)
    w2t = jnp.transpose(w2_hwio, (1, 3, 0, 2)).reshape(3 * planes, 3 * planes)
    w2t = w2t.astype(bf16)
    w3t = jnp.transpose(w3).astype(bf16)
    gb = jnp.stack([gammas[0], betas[0], gammas[1], betas[1],
                    gammas[2], betas[2]]).astype(f32)[:, :, None]

    cparams = pltpu.CompilerParams(
        dimension_semantics=("parallel",),
        vmem_limit_bytes=56 * 1024 * 1024)

    img_blk = pl.BlockSpec((G, C, HW), lambda n: (n, 0, 0))
    st_spec = pl.BlockSpec((1, C, 128), lambda n: (n, 0, 0))
    stfull_spec = pl.BlockSpec((ntiles, C, 128), lambda n: (0, 0, 0))
    gb_spec = pl.BlockSpec((6, C, 1), lambda n: (0, 0, 0))

    def wspec(shape):
        return pl.BlockSpec(shape, lambda n: (0, 0))

    h1, st1 = pl.pallas_call(
        partial(_conv1_kernel, g_imgs=G),
        grid=(ntiles,),
        in_specs=[img_blk, wspec((planes, C))],
        out_specs=[img_blk, st_spec],
        out_shape=[jax.ShapeDtypeStruct((N, planes, HW), bf16),
                   jax.ShapeDtypeStruct((ntiles, planes, 128), f32)],
        compiler_params=cparams,
    )(x3, w1t)

    h2, st2 = pl.pallas_call(
        partial(_conv2_kernel, g_imgs=G, c=planes, hw=HW, w_img=W, count=M),
        grid=(ntiles,),
        in_specs=[img_blk, stfull_spec, gb_spec,
                  wspec((3 * planes, 3 * planes))],
        out_specs=[img_blk, st_spec],
        out_shape=[jax.ShapeDtypeStruct((N, planes, HW), bf16),
                   jax.ShapeDtypeStruct((ntiles, planes, 128), f32)],
        compiler_params=cparams,
    )(h1, st1, gb, w2t)

    gram_spec = pl.BlockSpec((1, C, C), lambda n: (n, 0, 0))
    gramfull_spec = pl.BlockSpec((ntiles, C, C), lambda n: (0, 0, 0))
    sy3, gram = pl.pallas_call(
        partial(_conv3_stats_kernel, g_imgs=G, count=M),
        grid=(ntiles,),
        in_specs=[img_blk, stfull_spec, gb_spec],
        out_specs=[st_spec, gram_spec],
        out_shape=[jax.ShapeDtypeStruct((ntiles, planes, 128), f32),
                   jax.ShapeDtypeStruct((ntiles, planes, planes), f32)],
        compiler_params=cparams,
    )(h2, st2, gb)

    out3 = pl.pallas_call(
        partial(_final_kernel, g_imgs=G, count=M),
        grid=(ntiles,),
        in_specs=[img_blk, img_blk, stfull_spec, stfull_spec, gramfull_spec,
                  gb_spec, wspec((planes, planes))],
        out_specs=pl.BlockSpec((G, C, HW), lambda n: (n, 0, 0)),
        out_shape=jax.ShapeDtypeStruct((N, planes, HW), bf16),
        compiler_params=cparams,
    )(h2, x3, st2, sy3, gram, gb, w3t)

    return out3.reshape(N, planes, H, W).astype(f32)


# speedup vs baseline: 1.0051x; 1.0051x over previous
from functools import partial

import jax
import jax.numpy as jnp
from jax import lax
from jax.experimental import pallas as pl
from jax.experimental.pallas import tpu as pltpu

_EPS = 1e-5


def _bn_cols(st_ref, gb_ref, row, count):
    st = st_ref[...]
    s = jnp.sum(st[:, :, 0:1], axis=0)
    sq = jnp.sum(st[:, :, 1:2], axis=0)
    mean = s / count
    var = jnp.maximum(sq / count - mean * mean, 0.0)
    gamma = gb_ref[2 * row]
    beta = gb_ref[2 * row + 1]
    scale = gamma * lax.rsqrt(var + _EPS)
    shift = beta - mean * scale
    return scale, shift


def _accum(h, s_pl, q_pl):
    if s_pl is None:
        return h, h * h
    return s_pl + h, q_pl + h * h


def _store_stats(st_ref, s_pl, q_pl):
    st_ref[0, :, 0:1] = jnp.sum(s_pl, axis=1, keepdims=True)
    st_ref[0, :, 1:2] = jnp.sum(q_pl, axis=1, keepdims=True)


def _conv1_kernel(x_ref, w1t_ref, h1_ref, st_ref, *, g_imgs):
    f32 = jnp.float32
    s_pl = q_pl = None
    for g in range(g_imgs):
        h1 = jnp.dot(w1t_ref[...], x_ref[g], preferred_element_type=f32)
        h1_ref[g] = h1.astype(jnp.bfloat16)
        s_pl, q_pl = _accum(h1, s_pl, q_pl)
    _store_stats(st_ref, s_pl, q_pl)


def _conv2_kernel(h1_ref, st1_ref, gb_ref, w2_ref, h2_ref, st_ref, *,
                  g_imgs, c, hw, w_img, count):
    f32 = jnp.float32
    bf16 = jnp.bfloat16
    sc, sh = _bn_cols(st1_ref, gb_ref, 0, count)
    scb = sc.astype(bf16)
    shb = sh.astype(bf16)
    ww = lax.broadcasted_iota(jnp.int32, (1, hw), 1) % w_img
    ml = (ww >= 1).astype(f32)
    mr = (ww <= w_img - 2).astype(f32)
    zrow = jnp.zeros((c, w_img), bf16)
    z1 = jnp.zeros((c, 1), f32)
    s_pl = q_pl = None
    for g in range(g_imgs):
        yb = jnp.maximum(h1_ref[g] * scb + shb, 0)
        u_m = jnp.concatenate([zrow, yb[:, :hw - w_img]], axis=1)
        u_p = jnp.concatenate([yb[:, w_img:], zrow], axis=1)
        slab = jnp.concatenate([u_m, yb, u_p], axis=0)
        d_all = jnp.dot(w2_ref[...], slab, preferred_element_type=f32)
        d_m = d_all[0:c]
        d_0 = d_all[c:2 * c]
        d_p = d_all[2 * c:3 * c]
        h2 = (d_0
              + ml * jnp.concatenate([z1, d_m[:, :hw - 1]], axis=1)
              + mr * jnp.concatenate([d_p[:, 1:], z1], axis=1))
        h2_ref[g] = h2.astype(bf16)
        s_pl, q_pl = _accum(h2, s_pl, q_pl)
    _store_stats(st_ref, s_pl, q_pl)


def _conv3_stats_kernel(h2_ref, st2_ref, gb_ref, w3t_ref, st_ref, *,
                        g_imgs, count):
    f32 = jnp.float32
    bf16 = jnp.bfloat16
    sc, sh = _bn_cols(st2_ref, gb_ref, 1, count)
    scb = sc.astype(bf16)
    shb = sh.astype(bf16)
    s_pl = q_pl = None
    for g in range(g_imgs):
        y2 = jnp.maximum(h2_ref[g] * scb + shb, 0)
        h3 = jnp.dot(w3t_ref[...], y2, preferred_element_type=f32)
        s_pl, q_pl = _accum(h3, s_pl, q_pl)
    _store_stats(st_ref, s_pl, q_pl)


def _final_kernel(h2_ref, x_ref, st2_ref, st3_ref, gb_ref, w3t_ref, o_ref, *,
                  g_imgs, count):
    f32 = jnp.float32
    sc2, sh2 = _bn_cols(st2_ref, gb_ref, 1, count)
    scb2 = sc2.astype(jnp.bfloat16)
    shb2 = sh2.astype(jnp.bfloat16)
    sc3, sh3 = _bn_cols(st3_ref, gb_ref, 2, count)
    for g in range(g_imgs):
        y2 = jnp.maximum(h2_ref[g] * scb2 + shb2, 0)
        h3 = jnp.dot(w3t_ref[...], y2, preferred_element_type=f32)
        o = jnp.maximum(h3 * sc3 + sh3 + x_ref[g].astype(f32), 0.0)
        o_ref[g] = o.astype(jnp.bfloat16)


def kernel(x_nchw, w1, w2_hwio, w3, gammas, betas):
    N, C, H, W = x_nchw.shape
    planes = w1.shape[1]
    assert planes == C, "residual add requires planes == inplanes"
    HW = H * W
    M = float(N * HW)
    f32 = jnp.float32
    bf16 = jnp.bfloat16

    G = 16 if N % 16 == 0 else (8 if N % 8 == 0 else 1)
    ntiles = N // G

    x3 = x_nchw.reshape(N, C, HW).astype(bf16)

    w1t = jnp.transpose(w1).astype(bf16)
    w2t = jnp.transpose(w2_hwio, (1, 3, 0, 2)).reshape(3 * planes, 3 * planes)
    w2t = w2t.astype(bf16)
    w3t = jnp.transpose(w3).astype(bf16)
    gb = jnp.stack([gammas[0], betas[0], gammas[1], betas[1],
                    gammas[2], betas[2]]).astype(f32)[:, :, None]

    cparams = pltpu.CompilerParams(
        dimension_semantics=("parallel",),
        vmem_limit_bytes=56 * 1024 * 1024)

    img_blk = pl.BlockSpec((G, C, HW), lambda n: (n, 0, 0))
    st_spec = pl.BlockSpec((1, C, 128), lambda n: (n, 0, 0))
    stfull_spec = pl.BlockSpec((ntiles, C, 128), lambda n: (0, 0, 0))
    gb_spec = pl.BlockSpec((6, C, 1), lambda n: (0, 0, 0))

    def wspec(shape):
        return pl.BlockSpec(shape, lambda n: (0, 0))

    h1, st1 = pl.pallas_call(
        partial(_conv1_kernel, g_imgs=G),
        grid=(ntiles,),
        in_specs=[img_blk, wspec((planes, C))],
        out_specs=[img_blk, st_spec],
        out_shape=[jax.ShapeDtypeStruct((N, planes, HW), bf16),
                   jax.ShapeDtypeStruct((ntiles, planes, 128), f32)],
        compiler_params=cparams,
    )(x3, w1t)

    h2, st2 = pl.pallas_call(
        partial(_conv2_kernel, g_imgs=G, c=planes, hw=HW, w_img=W, count=M),
        grid=(ntiles,),
        in_specs=[img_blk, stfull_spec, gb_spec,
                  wspec((3 * planes, 3 * planes))],
        out_specs=[img_blk, st_spec],
        out_shape=[jax.ShapeDtypeStruct((N, planes, HW), bf16),
                   jax.ShapeDtypeStruct((ntiles, planes, 128), f32)],
        compiler_params=cparams,
    )(h1, st1, gb, w2t)

    st3 = pl.pallas_call(
        partial(_conv3_stats_kernel, g_imgs=G, count=M),
        grid=(ntiles,),
        in_specs=[img_blk, stfull_spec, gb_spec, wspec((planes, planes))],
        out_specs=st_spec,
        out_shape=jax.ShapeDtypeStruct((ntiles, planes, 128), f32),
        compiler_params=cparams,
    )(h2, st2, gb, w3t)

    out3 = pl.pallas_call(
        partial(_final_kernel, g_imgs=G, count=M),
        grid=(ntiles,),
        in_specs=[img_blk, img_blk, stfull_spec, stfull_spec, gb_spec,
                  wspec((planes, planes))],
        out_specs=pl.BlockSpec((G, C, HW), lambda n: (n, 0, 0)),
        out_shape=jax.ShapeDtypeStruct((N, planes, HW), bf16),
        compiler_params=cparams,
    )(h2, x3, st2, st3, gb, w3t)

    return out3.reshape(N, planes, H, W).astype(f32)
